# baseline (speedup 1.0000x reference)
import functools

import jax
import jax.numpy as jnp
import numpy as np
from jax import lax
from jax.experimental import pallas as pl
from jax.experimental.pallas import tpu as pltpu
from jax.experimental.pallas import tpu_sc as plsc

_NUM_GROUP = 4
_NTAU = 50
_N = 1048576
_L = 16
_INFO = plsc.get_sparse_core_info()
_NC = _INFO.num_cores
_NS = _INFO.num_subcores
_NW = _NC * _NS
_CHUNK = _N // _NW
_UNROLL = 16
_NSTEP = _CHUNK // (_L * _UNROLL)
_CNT_SHIFT = 18

_TAUS_PAD = np.full((64,), 2.0, dtype=np.float32)
_TAUS_PAD[0] = -1.0
_TAUS_PAD[1:_NTAU + 1] = np.arange(0.0, 1.0, 0.02).astype(np.float32)


@functools.partial(
    pl.kernel,
    out_type=jax.ShapeDtypeStruct((_NW, 64), jnp.int32),
    mesh=plsc.VectorSubcoreMesh(core_axis_name="c", subcore_axis_name="s"),
    compiler_params=pltpu.CompilerParams(needs_layout_passes=False),
    scratch_types=[
        pltpu.VMEM((_CHUNK,), jnp.float32),
        pltpu.VMEM((_CHUNK,), jnp.int32),
        pltpu.VMEM((64,), jnp.float32),
        pltpu.VMEM((64,), jnp.int32),
        pltpu.SemaphoreType.DMA,
        pltpu.SemaphoreType.DMA,
    ],
)
def _spdd_sc(pred_hbm, group_hbm, tau_hbm, out_hbm, pred_v, grp_v, tau_v, acc_v,
             sem_p, sem_g):
    wid = lax.axis_index("s") * _NC + lax.axis_index("c")
    base = wid * _CHUNK
    cp_p = pltpu.async_copy(pred_hbm.at[pl.ds(base, _CHUNK)], pred_v, sem_p)
    cp_g = pltpu.async_copy(group_hbm.at[pl.ds(base, _CHUNK)], grp_v, sem_g)
    pltpu.sync_copy(tau_hbm, tau_v)

    zeros16 = jnp.zeros((_L,), jnp.int32)
    for j in range(64 // _L):
        acc_v[pl.ds(j * _L, _L)] = zeros16
    cp_p.wait()
    cp_g.wait()

    lane = lax.iota(jnp.int32, _L)
    pack_one = jnp.full((_L,), 1 << _CNT_SHIFT, jnp.int32)

    @plsc.parallel_loop(0, _CHUNK // _L, 1, unroll=_UNROLL)
    def _(i):
        off = i * _L
        p = pred_v[pl.ds(off, _L)]
        g = grp_v[pl.ds(off, _L)]
        k0 = (p * 50.0).astype(jnp.int32)
        t0 = plsc.load_gather(tau_v, [k0 + 1])
        t1 = plsc.load_gather(tau_v, [k0 + 2])
        c = k0 + jnp.where(p > t0, 1, 0) + jnp.where(p > t1, 1, 0)
        idx = (g << 4) + lane
        plsc.addupdate_scatter(acc_v, [idx], c + pack_one)

    pltpu.sync_copy(acc_v, out_hbm.at[wid])


def kernel(pred, target, group):
    del target
    parts = _spdd_sc(pred, group, jnp.asarray(_TAUS_PAD))
    csum = (parts & ((1 << _CNT_SHIFT) - 1)).sum(axis=0)
    cnt = (parts >> _CNT_SHIFT).sum(axis=0)
    c_tot = csum.reshape(_NUM_GROUP, _L).sum(axis=1).astype(jnp.float32)
    n_tot = cnt.reshape(_NUM_GROUP, _L).sum(axis=1).astype(jnp.float32)
    par = c_tot / (n_tot + 1e-10) / np.float32(_NTAU)
    iu, ju = np.triu_indices(_NUM_GROUP, 1)
    disp = jnp.abs(par[iu] - par[ju])
    return (disp.mean(), disp.max())


# speedup vs baseline: 1.0951x; 1.0951x over previous
import functools

import jax
import jax.numpy as jnp
import numpy as np
from jax import lax
from jax.experimental import pallas as pl
from jax.experimental.pallas import tpu as pltpu
from jax.experimental.pallas import tpu_sc as plsc

_NUM_GROUP = 4
_NTAU = 50
_N = 1048576
_L = 16
_INFO = plsc.get_sparse_core_info()
_NC = _INFO.num_cores
_NS = _INFO.num_subcores
_NW = _NC * _NS
_CHUNK = _N // _NW
_UNROLL = 8
_NSTEP = _CHUNK // (_L * _UNROLL)
_CNT_SHIFT = 18

_TAUS_PAD = np.full((64,), 2.0, dtype=np.float32)
_TAUS_PAD[0] = -1.0
_TAUS_PAD[1:_NTAU + 1] = np.arange(0.0, 1.0, 0.02).astype(np.float32)


@functools.partial(
    pl.kernel,
    out_type=jax.ShapeDtypeStruct((_NW, 64), jnp.int32),
    mesh=plsc.VectorSubcoreMesh(core_axis_name="c", subcore_axis_name="s"),
    compiler_params=pltpu.CompilerParams(needs_layout_passes=False),
    scratch_types=[
        pltpu.VMEM((_CHUNK,), jnp.float32),
        pltpu.VMEM((_CHUNK,), jnp.int32),
        pltpu.VMEM((64,), jnp.float32),
        pltpu.VMEM((64,), jnp.int32),
        pltpu.SemaphoreType.DMA,
        pltpu.SemaphoreType.DMA,
        pltpu.SemaphoreType.DMA,
    ],
)
def _spdd_sc(pred_hbm, group_hbm, tau_hbm, out_hbm, pred_v, grp_v, tau_v, acc_v,
             sem_p, sem_g, sem_t):
    wid = lax.axis_index("s") * _NC + lax.axis_index("c")
    base = wid * _CHUNK
    cp_p = pltpu.async_copy(pred_hbm.at[pl.ds(base, _CHUNK)], pred_v, sem_p)
    cp_g = pltpu.async_copy(group_hbm.at[pl.ds(base, _CHUNK)], grp_v, sem_g)
    cp_t = pltpu.async_copy(tau_hbm, tau_v, sem_t)

    zeros16 = jnp.zeros((_L,), jnp.int32)
    for j in range(64 // _L):
        acc_v[pl.ds(j * _L, _L)] = zeros16
    cp_t.wait()
    cp_p.wait()
    cp_g.wait()

    lane = lax.iota(jnp.int32, _L)
    pack_one = jnp.full((_L,), 1 << _CNT_SHIFT, jnp.int32)

    @plsc.parallel_loop(0, _CHUNK // _L, 1, unroll=_UNROLL)
    def _(i):
        off = i * _L
        p = pred_v[pl.ds(off, _L)]
        g = grp_v[pl.ds(off, _L)]
        k0 = (p * 50.0).astype(jnp.int32)
        t0 = plsc.load_gather(tau_v, [k0 + 1])
        t1 = plsc.load_gather(tau_v, [k0 + 2])
        c = k0 + jnp.where(p > t0, 1, 0) + jnp.where(p > t1, 1, 0)
        idx = (g << 4) + lane
        plsc.addupdate_scatter(acc_v, [idx], c + pack_one)

    pltpu.sync_copy(acc_v, out_hbm.at[wid])


def kernel(pred, target, group):
    del target
    parts = _spdd_sc(pred, group, jnp.asarray(_TAUS_PAD))
    csum = (parts & ((1 << _CNT_SHIFT) - 1)).sum(axis=0)
    cnt = (parts >> _CNT_SHIFT).sum(axis=0)
    c_tot = csum.reshape(_NUM_GROUP, _L).sum(axis=1).astype(jnp.float32)
    n_tot = cnt.reshape(_NUM_GROUP, _L).sum(axis=1).astype(jnp.float32)
    par = c_tot / (n_tot + 1e-10) / np.float32(_NTAU)
    iu, ju = np.triu_indices(_NUM_GROUP, 1)
    disp = jnp.abs(par[iu] - par[ju])
    return (disp.mean(), disp.max())


# speedup vs baseline: 1.0963x; 1.0010x over previous
import functools

import jax
import jax.numpy as jnp
import numpy as np
from jax import lax
from jax.experimental import pallas as pl
from jax.experimental.pallas import tpu as pltpu
from jax.experimental.pallas import tpu_sc as plsc

_NUM_GROUP = 4
_NTAU = 50
_N = 1048576
_L = 16
_INFO = plsc.get_sparse_core_info()
_NC = _INFO.num_cores
_NS = _INFO.num_subcores
_NW = _NC * _NS
_CHUNK = _N // _NW
_UNROLL = 8
_NSTEP = _CHUNK // (_L * _UNROLL)
_CNT_SHIFT = 18

_TAUS_PAD = np.full((64,), 2.0, dtype=np.float32)
_TAUS_PAD[0] = -1.0
_TAUS_PAD[1:_NTAU + 1] = np.arange(0.0, 1.0, 0.02).astype(np.float32)


@functools.partial(
    pl.kernel,
    out_type=jax.ShapeDtypeStruct((_NW, 64), jnp.int32),
    mesh=plsc.VectorSubcoreMesh(core_axis_name="c", subcore_axis_name="s"),
    compiler_params=pltpu.CompilerParams(needs_layout_passes=False),
    scratch_types=[
        pltpu.VMEM((_CHUNK,), jnp.float32),
        pltpu.VMEM((_CHUNK,), jnp.int32),
        pltpu.VMEM((64,), jnp.float32),
        pltpu.VMEM((64,), jnp.int32),
        pltpu.SemaphoreType.DMA,
        pltpu.SemaphoreType.DMA,
    ],
)
def _spdd_sc(pred_hbm, group_hbm, tau_hbm, out_hbm, pred_v, grp_v, tau_v, acc_v,
             sem_p, sem_g):
    wid = lax.axis_index("s") * _NC + lax.axis_index("c")
    base = wid * _CHUNK
    cp_p = pltpu.async_copy(pred_hbm.at[pl.ds(base, _CHUNK)], pred_v, sem_p)
    cp_g = pltpu.async_copy(group_hbm.at[pl.ds(base, _CHUNK)], grp_v, sem_g)
    pltpu.sync_copy(tau_hbm, tau_v)

    zeros16 = jnp.zeros((_L,), jnp.int32)
    for j in range(64 // _L):
        acc_v[pl.ds(j * _L, _L)] = zeros16
    cp_p.wait()
    cp_g.wait()
    plsc.subcore_barrier()

    lane = lax.iota(jnp.int32, _L)
    pack_one = jnp.full((_L,), 1 << _CNT_SHIFT, jnp.int32)

    @plsc.parallel_loop(0, _CHUNK // _L, 1, unroll=_UNROLL)
    def _(i):
        off = i * _L
        p = pred_v[pl.ds(off, _L)]
        g = grp_v[pl.ds(off, _L)]
        k0 = (p * 50.0).astype(jnp.int32)
        t0 = plsc.load_gather(tau_v, [k0 + 1])
        t1 = plsc.load_gather(tau_v, [k0 + 2])
        c = k0 + jnp.where(p > t0, 1, 0) + jnp.where(p > t1, 1, 0)
        idx = (g << 4) + lane
        plsc.addupdate_scatter(acc_v, [idx], c + pack_one)

    pltpu.sync_copy(acc_v, out_hbm.at[wid])


def kernel(pred, target, group):
    del target
    parts = _spdd_sc(pred, group, jnp.asarray(_TAUS_PAD))
    csum = (parts & ((1 << _CNT_SHIFT) - 1)).sum(axis=0)
    cnt = (parts >> _CNT_SHIFT).sum(axis=0)
    c_tot = csum.reshape(_NUM_GROUP, _L).sum(axis=1).astype(jnp.float32)
    n_tot = cnt.reshape(_NUM_GROUP, _L).sum(axis=1).astype(jnp.float32)
    par = c_tot / (n_tot + 1e-10) / np.float32(_NTAU)
    iu, ju = np.triu_indices(_NUM_GROUP, 1)
    disp = jnp.abs(par[iu] - par[ju])
    return (disp.mean(), disp.max())
